# baseline (speedup 1.0000x reference)
import jax
import jax.numpy as jnp
from jax import lax
from jax.experimental import pallas as pl
from jax.experimental.pallas import tpu as pltpu
from jax.experimental.pallas import tpu_sc as plsc

N = 100000
D = 128
B = 512
NEG_INF = float("-inf")

NSC = 2
NTEC = 16
CHUNK = 6256
TAIL_SKIP = (NTEC - 1) * CHUNK - (N - CHUNK)
SLAB = 192
NSLAB = (CHUNK + SLAB - 1) // SLAB
SEG_PER_TILE = B // NTEC
NG = D // 16


def _sc_body(x_hbm, ids_hbm, part, part_cnt,
             ids_v, xs_v, acc, cnt_acc, mask_v, sems):
    cid = lax.axis_index("c")
    ts = lax.axis_index("s")
    tail = ts // (NTEC - 1)
    row_base = ts * CHUNK - tail * TAIL_SKIP

    zero16 = jnp.zeros((16,), jnp.float32)
    ninf16 = jnp.full((16,), NEG_INF, jnp.float32)
    ones16 = jnp.full((16,), 1.0, jnp.float32)

    @pl.when(cid == 0)
    def _mask_sum():
        mask_v[pl.ds(0, 16)] = ones16

    @pl.when(cid == 1)
    def _mask_max():
        mask_v[pl.ds(0, 16)] = zero16

    def _is_sum16():
        return mask_v[pl.ds(0, 16)] > 0.5

    @pl.when(cid == 0)
    def _init_sum():
        def init_body(r, _):
            for g in range(NG):
                acc[r, pl.ds(g * 16, 16)] = zero16
            return 0

        lax.fori_loop(0, B, init_body, 0)

    @pl.when(cid == 1)
    def _init_max():
        def init_body(r, _):
            for g in range(NG):
                acc[r, pl.ds(g * 16, 16)] = ninf16
            return 0

        lax.fori_loop(0, B, init_body, 0)

    def init_cnt(i, _):
        cnt_acc[pl.ds(i * 16, 16)] = zero16
        return 0

    lax.fori_loop(0, B, init_cnt, 0)

    lo0 = tail * TAIL_SKIP

    def _src_base(sl):
        return jnp.minimum(row_base + sl * SLAB, N - SLAB)

    def _slab_src(sl):
        return x_hbm.at[pl.ds(_src_base(sl) * D, SLAB * D)]

    def _ids_src(sl):
        return ids_hbm.at[pl.ds(_src_base(sl), SLAB)]

    def _fold(vals, op):
        v = op(vals[0], vals[1])
        for x in vals[2:]:
            v = op(v, x)
        return v

    def group_body(k, sl):
        buf = sl % 2
        shift = row_base + sl * SLAB - _src_base(sl)
        rr0 = pl.multiple_of(
            lo0 * jnp.maximum(0, 1 - sl) + k * 16 + shift, 16)
        seg16 = ids_v[pl.ds(pl.multiple_of(buf * 256 + rr0, 16), 16)]
        s_first = seg16[0]
        s_last = seg16[15]

        xbase = pl.multiple_of(rr0 * D, 16)

        def _xoff(j, g):
            return xbase + (j * D + g * 16)

        @pl.when((s_first == s_last) & (cid == 0))
        def _uniform_sum():
            for g in range(NG):
                cs = pl.ds(g * 16, 16)
                vals = [xs_v[buf, pl.ds(_xoff(j, g), 16)]
                        for j in range(16)]
                vsum = _fold(vals, lambda a, b: a + b)
                acc[s_first, cs] = acc[s_first, cs] + vsum
            c16 = pl.ds(pl.multiple_of(s_first * 16, 16), 16)
            cnt_acc[c16] = cnt_acc[c16] + ones16 * 16.0

        @pl.when((s_first == s_last) & (cid == 1))
        def _uniform_max():
            for g in range(NG):
                cs = pl.ds(g * 16, 16)
                vals = [xs_v[buf, pl.ds(_xoff(j, g), 16)]
                        for j in range(16)]
                vmax = _fold(vals, jnp.maximum)
                acc[s_first, cs] = jnp.maximum(acc[s_first, cs], vmax)

        @pl.when(s_first != s_last)
        def _boundary():
            for j in range(16):
                seg = seg16[j]
                for g in range(NG):
                    cs = pl.ds(g * 16, 16)
                    v = xs_v[buf, pl.ds(_xoff(j, g), 16)]
                    a = acc[seg, cs]
                    acc[seg, cs] = jnp.where(
                        _is_sum16(), a + v, jnp.maximum(a, v))
                c16 = pl.ds(pl.multiple_of(seg * 16, 16), 16)
                cnt_acc[c16] = cnt_acc[c16] + ones16

        return sl

    def make_slab_body(group_body):
        def slab_body(sl, _):
            parity = sl % 2

            @pl.when(sl + 1 < NSLAB)
            def _prefetch():
                pltpu.async_copy(_slab_src(sl + 1), xs_v.at[1 - parity],
                                 sems.at[1 - parity])
                pltpu.async_copy(_ids_src(sl + 1),
                                 ids_v.at[pl.ds((1 - parity) * 256, SLAB)],
                                 sems.at[3 - parity])

            pltpu.make_async_copy(_slab_src(sl), xs_v.at[parity],
                                  sems.at[parity]).wait()
            pltpu.make_async_copy(_ids_src(sl),
                                  ids_v.at[pl.ds(parity * 256, SLAB)],
                                  sems.at[2 + parity]).wait()
            ngroups = (jnp.minimum((sl + 1) * SLAB, CHUNK) - sl * SLAB
                       - lo0 * jnp.maximum(0, 1 - sl)) // 16
            lax.fori_loop(0, ngroups, group_body, sl)
            return 0

        return slab_body

    pltpu.async_copy(_slab_src(0), xs_v.at[0], sems.at[0])
    pltpu.async_copy(_ids_src(0), ids_v.at[pl.ds(0, SLAB)], sems.at[2])

    lax.fori_loop(0, NSLAB, make_slab_body(group_body), 0)

    pltpu.sync_copy(acc, part.at[cid * NTEC + ts])

    @pl.when(cid == 0)
    def _pub_cnt():
        pltpu.sync_copy(cnt_acc, part_cnt.at[ts])


def _sc_aggregate(x_flat, ids):
    mesh = plsc.VectorSubcoreMesh(
        core_axis_name="c", subcore_axis_name="s",
        num_cores=NSC, num_subcores=NTEC)
    f = pl.kernel(
        _sc_body,
        out_type=(
            jax.ShapeDtypeStruct((NSC * NTEC, B, D), jnp.float32),
            jax.ShapeDtypeStruct((NTEC, B * 16), jnp.float32),
        ),
        mesh=mesh,
        scratch_types=[
            pltpu.VMEM((512,), jnp.int32),
            pltpu.VMEM((2, SLAB * D), jnp.float32),
            pltpu.VMEM((B, D), jnp.float32),
            pltpu.VMEM((B * 16,), jnp.float32),
            pltpu.VMEM((16,), jnp.float32),
            pltpu.SemaphoreType.DMA((4,)),
        ],
    )
    return f(x_flat, ids)


def _proj_body(part_ref, cntp_ref, w1t_ref, w2t_ref, bias_ref, out_ref):
    ps = part_ref[...]
    sums = jnp.sum(ps[:NTEC], axis=0)
    maxs = jnp.max(ps[NTEC:], axis=0)
    cnt = jnp.sum(cntp_ref[...], axis=0)[:, :1]
    mean = sums / jnp.maximum(cnt, 1.0)
    out_ref[...] = (
        jax.lax.dot(mean, w1t_ref[...],
                    precision=jax.lax.Precision.HIGHEST,
                    preferred_element_type=jnp.float32)
        + jax.lax.dot(maxs, w2t_ref[...],
                      precision=jax.lax.Precision.HIGHEST,
                      preferred_element_type=jnp.float32)
        + bias_ref[...])


@jax.jit
def kernel(x, batch, W, b):
    ids = batch.astype(jnp.int32)
    part, part_cnt = _sc_aggregate(x.reshape(-1), ids)
    w1t = W[:, :D].T
    w2t = W[:, D:].T
    bias = b.reshape(1, D)
    out = pl.pallas_call(
        _proj_body,
        out_shape=jax.ShapeDtypeStruct((B, D), jnp.float32),
    )(part, part_cnt.reshape(NTEC, B, 16), w1t, w2t, bias)
    return out


# speedup vs baseline: 1.0485x; 1.0485x over previous
import jax
import jax.numpy as jnp
from jax import lax
from jax.experimental import pallas as pl
from jax.experimental.pallas import tpu as pltpu
from jax.experimental.pallas import tpu_sc as plsc

N = 100000
D = 128
B = 512
NEG_INF = float("-inf")

NSC = 2
NTEC = 16
CHUNK = 6256
TAIL_SKIP = (NTEC - 1) * CHUNK - (N - CHUNK)
SLAB = 192
NSLAB = (CHUNK + SLAB - 1) // SLAB
SEG_PER_TILE = B // NTEC
NG = D // 16


def _sc_body(x_hbm, ids_hbm, part, part_cnt,
             ids_v, xs_v, acc, cnt_acc, mask_v, sems):
    cid = lax.axis_index("c")
    ts = lax.axis_index("s")
    tail = ts // (NTEC - 1)
    row_base = ts * CHUNK - tail * TAIL_SKIP

    zero16 = jnp.zeros((16,), jnp.float32)
    ninf16 = jnp.full((16,), NEG_INF, jnp.float32)
    ones16 = jnp.full((16,), 1.0, jnp.float32)

    @pl.when(cid == 0)
    def _mask_sum():
        mask_v[pl.ds(0, 16)] = ones16

    @pl.when(cid == 1)
    def _mask_max():
        mask_v[pl.ds(0, 16)] = zero16

    def _is_sum16():
        return mask_v[pl.ds(0, 16)] > 0.5

    @pl.when(cid == 0)
    def _init_sum():
        def init_body(r, _):
            for g in range(NG):
                acc[r, pl.ds(g * 16, 16)] = zero16
            return 0

        lax.fori_loop(0, B, init_body, 0)

    @pl.when(cid == 1)
    def _init_max():
        def init_body(r, _):
            for g in range(NG):
                acc[r, pl.ds(g * 16, 16)] = ninf16
            return 0

        lax.fori_loop(0, B, init_body, 0)

    def init_cnt(i, _):
        cnt_acc[pl.ds(i * 16, 16)] = zero16
        return 0

    lax.fori_loop(0, B, init_cnt, 0)

    lo0 = tail * TAIL_SKIP

    def _src_base(sl):
        return jnp.minimum(row_base + sl * SLAB, N - SLAB)

    def _slab_src(sl):
        return x_hbm.at[pl.ds(_src_base(sl) * D, SLAB * D)]

    def _ids_src(sl):
        return ids_hbm.at[pl.ds(_src_base(sl), SLAB)]

    def _fold(vals, op):
        v = op(vals[0], vals[1])
        for x in vals[2:]:
            v = op(v, x)
        return v

    def group_body(k, sl):
        buf = sl % 2
        shift = row_base + sl * SLAB - _src_base(sl)
        rr0 = pl.multiple_of(
            lo0 * jnp.maximum(0, 1 - sl) + k * 16 + shift, 16)
        seg16 = ids_v[pl.ds(pl.multiple_of(buf * 256 + rr0, 16), 16)]
        s_first = seg16[0]
        s_last = seg16[15]

        xbase = pl.multiple_of(rr0 * D, 16)

        def _xoff(j, g):
            return xbase + (j * D + g * 16)

        @pl.when(s_first == s_last)
        def _uniform():
            for g in range(NG):
                cs = pl.ds(g * 16, 16)
                vals = [xs_v[buf, pl.ds(_xoff(j, g), 16)]
                        for j in range(16)]
                vsum = _fold(vals, lambda a, b: a + b)
                vmax = _fold(vals, jnp.maximum)
                a = acc[s_first, cs]
                acc[s_first, cs] = jnp.where(
                    _is_sum16(), a + vsum, jnp.maximum(a, vmax))
            c16 = pl.ds(pl.multiple_of(s_first * 16, 16), 16)
            cnt_acc[c16] = cnt_acc[c16] + ones16 * 16.0

        @pl.when(s_first != s_last)
        def _boundary():
            for j in range(16):
                seg = seg16[j]
                for g in range(NG):
                    cs = pl.ds(g * 16, 16)
                    v = xs_v[buf, pl.ds(_xoff(j, g), 16)]
                    a = acc[seg, cs]
                    acc[seg, cs] = jnp.where(
                        _is_sum16(), a + v, jnp.maximum(a, v))
                c16 = pl.ds(pl.multiple_of(seg * 16, 16), 16)
                cnt_acc[c16] = cnt_acc[c16] + ones16

        return sl

    def make_slab_body(group_body):
        def slab_body(sl, _):
            parity = sl % 2

            @pl.when(sl + 1 < NSLAB)
            def _prefetch():
                pltpu.async_copy(_slab_src(sl + 1), xs_v.at[1 - parity],
                                 sems.at[1 - parity])
                pltpu.async_copy(_ids_src(sl + 1),
                                 ids_v.at[pl.ds((1 - parity) * 256, SLAB)],
                                 sems.at[3 - parity])

            pltpu.make_async_copy(_slab_src(sl), xs_v.at[parity],
                                  sems.at[parity]).wait()
            pltpu.make_async_copy(_ids_src(sl),
                                  ids_v.at[pl.ds(parity * 256, SLAB)],
                                  sems.at[2 + parity]).wait()
            ngroups = (jnp.minimum((sl + 1) * SLAB, CHUNK) - sl * SLAB
                       - lo0 * jnp.maximum(0, 1 - sl)) // 16
            lax.fori_loop(0, ngroups, group_body, sl)
            return 0

        return slab_body

    pltpu.async_copy(_slab_src(0), xs_v.at[0], sems.at[0])
    pltpu.async_copy(_ids_src(0), ids_v.at[pl.ds(0, SLAB)], sems.at[2])

    lax.fori_loop(0, NSLAB, make_slab_body(group_body), 0)

    pltpu.sync_copy(acc, part.at[cid * NTEC + ts])

    @pl.when(cid == 0)
    def _pub_cnt():
        pltpu.sync_copy(cnt_acc, part_cnt.at[ts])


def _sc_aggregate(x_flat, ids):
    mesh = plsc.VectorSubcoreMesh(
        core_axis_name="c", subcore_axis_name="s",
        num_cores=NSC, num_subcores=NTEC)
    f = pl.kernel(
        _sc_body,
        out_type=(
            jax.ShapeDtypeStruct((NSC * NTEC, B, D), jnp.float32),
            jax.ShapeDtypeStruct((NTEC, B * 16), jnp.float32),
        ),
        mesh=mesh,
        scratch_types=[
            pltpu.VMEM((512,), jnp.int32),
            pltpu.VMEM((2, SLAB * D), jnp.float32),
            pltpu.VMEM((B, D), jnp.float32),
            pltpu.VMEM((B * 16,), jnp.float32),
            pltpu.VMEM((16,), jnp.float32),
            pltpu.SemaphoreType.DMA((4,)),
        ],
    )
    return f(x_flat, ids)


def _proj_body(part_ref, cntp_ref, w1t_ref, w2t_ref, bias_ref, out_ref):
    ps = part_ref[...]
    sums = jnp.sum(ps[:NTEC], axis=0)
    maxs = jnp.max(ps[NTEC:], axis=0)
    cnt = jnp.sum(cntp_ref[...], axis=0)[:, :1]
    mean = sums / jnp.maximum(cnt, 1.0)
    out_ref[...] = (
        jax.lax.dot(mean, w1t_ref[...],
                    precision=jax.lax.Precision.HIGHEST,
                    preferred_element_type=jnp.float32)
        + jax.lax.dot(maxs, w2t_ref[...],
                      precision=jax.lax.Precision.HIGHEST,
                      preferred_element_type=jnp.float32)
        + bias_ref[...])


@jax.jit
def kernel(x, batch, W, b):
    ids = batch.astype(jnp.int32)
    part, part_cnt = _sc_aggregate(x.reshape(-1), ids)
    w1t = W[:, :D].T
    w2t = W[:, D:].T
    bias = b.reshape(1, D)
    out = pl.pallas_call(
        _proj_body,
        out_shape=jax.ShapeDtypeStruct((B, D), jnp.float32),
    )(part, part_cnt.reshape(NTEC, B, 16), w1t, w2t, bias)
    return out
